# baseline (speedup 1.0000x reference)
import jax
import jax.numpy as jnp
from jax.experimental import pallas as pl
from jax.experimental.pallas import tpu as pltpu

_PLANES = 16 * 32 * 3 * 7
_GROUPS = _PLANES // 21
_BG = 32
_BP = 21 * _BG
_GRID = _PLANES // _BP


def _body(out_ref, lab_ref, res_ref, acc_ref):
    pid = pl.program_id(0)

    @pl.when(pid == 0)
    def _init():
        acc_ref[...] = jnp.zeros_like(acc_ref)

    z = jnp.zeros((32, 32), jnp.float32)
    part = [z] * 12

    for g in range(_BG):
        for c in range(3):
            p0 = 21 * g + 7 * c
            conf = lab_ref[p0]
            o0 = out_ref[p0]
            pos = jnp.where(conf > 0.5, 1.0, 0.0)
            neg = jnp.where(conf < -0.5, 1.0, 0.0)
            a = jnp.abs(o0)
            base = jnp.log1p(jnp.exp(-a))
            r = jnp.maximum(o0, 0.0)
            part[0] += pos * (base + (a - r))
            part[1] += neg * (base + r)
            part[2] += pos
            part[3] += neg
            ge = o0 >= 0.0
            part[4] += jnp.where(ge, pos, 0.0)
            part[5] += jnp.where(ge, 0.0, neg)
            for f in range(1, 7):
                d = out_ref[p0 + f] - lab_ref[p0 + f]
                ad = jnp.abs(d)
                m = jnp.minimum(ad, 1.0)
                part[5 + f] += pos * (m * (ad - 0.5 * m))

    for q in range(12):
        acc_ref[q] += part[q]

    @pl.when(pid == _GRID - 1)
    def _final():
        sums = [jnp.sum(acc_ref[q]) for q in range(12)]
        pb, nb, n_pos, n_neg, pc, nc = sums[:6]
        classify = 0.5 * pb / n_pos + 0.5 * nb / n_neg
        regs = [sums[5 + f] / n_pos for f in range(1, 7)]
        loss = classify
        for rv in regs:
            loss = loss + rv
        vals = [loss, classify] + regs + [pc, n_pos, nc, n_neg]
        for i, v in enumerate(vals):
            res_ref[i] = v


def kernel(output, labels):
    o3 = output.transpose(0, 1, 4, 5, 2, 3).reshape(_PLANES, 32, 32)
    t3 = labels.transpose(0, 1, 4, 5, 2, 3).reshape(_PLANES, 32, 32)
    res = pl.pallas_call(
        _body,
        grid=(_GRID,),
        in_specs=[
            pl.BlockSpec((_BP, 32, 32), lambda i: (i, 0, 0)),
            pl.BlockSpec((_BP, 32, 32), lambda i: (i, 0, 0)),
        ],
        out_specs=pl.BlockSpec(memory_space=pltpu.SMEM),
        out_shape=jax.ShapeDtypeStruct((12,), jnp.float32),
        scratch_shapes=[pltpu.VMEM((12, 32, 32), jnp.float32)],
        compiler_params=pltpu.CompilerParams(
            dimension_semantics=("arbitrary",)),
    )(o3, t3)
    return tuple(res[i] for i in range(12))
